# baseline (speedup 1.0000x reference)
import functools

import jax
import jax.numpy as jnp
from jax import lax
from jax.experimental import pallas as pl
from jax.experimental.pallas import tpu as pltpu
from jax.experimental.pallas import tpu_sc as plsc

NUM_SAMPLED = 100
VOCAB = 1000000
DIM = 32
BATCH = 4096

_S_PAD = 256
_S_COLS = 128


def _log_uniform_prob(classes_f32, range_max):
    return (jnp.log(classes_f32 + 2.0) - jnp.log(classes_f32 + 1.0)) / jnp.log(
        range_max + 1.0
    )


def _make_sc_gather(n_lab, n_samp, dim, nc, ns):
    nw = nc * ns
    lab_per_w = n_lab // nw
    samp_per_w = n_samp // nw
    mesh = plsc.VectorSubcoreMesh(core_axis_name="c", subcore_axis_name="s")

    @functools.partial(
        pl.kernel,
        mesh=mesh,
        compiler_params=pltpu.CompilerParams(use_tc_tiling_on_sc=False),
        out_type=jax.ShapeDtypeStruct((n_lab + n_samp, dim), jnp.float32),
        scratch_types=[
            pltpu.VMEM((lab_per_w,), jnp.int32),
            pltpu.VMEM((lab_per_w, dim), jnp.float32),
            pltpu.VMEM((samp_per_w,), jnp.int32),
            pltpu.VMEM((samp_per_w, dim), jnp.float32),
            pltpu.SemaphoreType.DMA,
            pltpu.SemaphoreType.DMA,
        ],
    )
    def sc_gather(table_hbm, idx_hbm, out_hbm, idx_l, rows_l, idx_s, rows_s,
                  sem_l, sem_s):
        wid = lax.axis_index("s") * nc + lax.axis_index("c")
        base_l = wid * lab_per_w
        base_s = n_lab + wid * samp_per_w
        pltpu.sync_copy(idx_hbm.at[pl.ds(base_l, lab_per_w)], idx_l)
        pltpu.sync_copy(idx_hbm.at[pl.ds(base_s, samp_per_w)], idx_s)
        g_l = pltpu.async_copy(table_hbm.at[idx_l], rows_l, sem_l)
        g_s = pltpu.async_copy(table_hbm.at[idx_s], rows_s, sem_s)
        g_l.wait()
        pltpu.sync_copy(rows_l, out_hbm.at[pl.ds(base_l, lab_per_w)])
        g_s.wait()
        pltpu.sync_copy(rows_s, out_hbm.at[pl.ds(base_s, samp_per_w)])

    return sc_gather


def _tc_body(user_ref, truew_ref, sampw_ref, labels_ref, sidx_ref, slog_ref,
             out_ref):
    u = user_ref[...]
    tw = truew_ref[...]
    sw = sampw_ref[...]
    lab = labels_ref[...]
    sidx = sidx_ref[...]
    slog = slog_ref[...]

    lf = lab.astype(jnp.float32)
    p_true = _log_uniform_prob(lf, float(VOCAB))
    true_logit = (
        jnp.sum(u * tw, axis=1, keepdims=True)
        - jnp.log(NUM_SAMPLED * p_true)
    )

    s_logits = (
        lax.dot_general(u, sw, (((1,), (1,)), ((), ())),
                        preferred_element_type=jnp.float32)
        - slog
    )
    col = lax.broadcasted_iota(jnp.int32, (1, _S_COLS), 1)
    dead = (sidx == lab) | (col >= NUM_SAMPLED)
    s_logits = jnp.where(dead, jnp.float32(-1e9), s_logits)

    m = jnp.maximum(jnp.max(s_logits, axis=1, keepdims=True), true_logit)
    ssum = jnp.sum(jnp.exp(s_logits - m), axis=1, keepdims=True) + jnp.exp(
        true_logit - m
    )
    out_ref[...] = m + jnp.log(ssum) - true_logit


def kernel(item_embeddings, user_embeddings, label_idx, zero_bias):
    del zero_bias
    table = item_embeddings.reshape(VOCAB, DIM)
    user = user_embeddings.reshape(BATCH, DIM)
    labels = label_idx.reshape(BATCH).astype(jnp.int32)

    skey = jax.random.key(42)
    u01 = jax.random.uniform(skey, (NUM_SAMPLED,), dtype=jnp.float32)
    sampled = jnp.clip(
        (jnp.exp(u01 * jnp.log(VOCAB + 1.0)) - 1.0).astype(labels.dtype),
        0,
        VOCAB - 1,
    )
    p_samp = _log_uniform_prob(sampled.astype(jnp.float32), float(VOCAB))
    slog = jnp.zeros((1, _S_COLS), jnp.float32).at[0, :NUM_SAMPLED].set(
        jnp.log(NUM_SAMPLED * p_samp)
    )
    sidx = jnp.full((1, _S_COLS), -1, jnp.int32).at[0, :NUM_SAMPLED].set(sampled)

    info = plsc.get_sparse_core_info()
    nc, ns = info.num_cores, info.num_subcores
    idx_all = jnp.concatenate(
        [labels, jnp.zeros((_S_PAD,), jnp.int32).at[:NUM_SAMPLED].set(sampled)]
    )
    gathered = _make_sc_gather(BATCH, _S_PAD, DIM, nc, ns)(table, idx_all)
    true_w = gathered[:BATCH]
    samp_w = gathered[BATCH : BATCH + _S_COLS]

    loss = pl.pallas_call(
        _tc_body,
        out_shape=jax.ShapeDtypeStruct((BATCH, 1), jnp.float32),
    )(user, true_w, samp_w, labels.reshape(BATCH, 1), sidx, slog)
    return loss
